# Initial kernel scaffold; baseline (speedup 1.0000x reference)
import jax
import jax.numpy as jnp
from jax.experimental import pallas as pl


def kernel(x_nchw, conv_weight):
    raise NotImplementedError("write your pallas kernel here")


# speedup vs baseline: 1.0244x; 1.0244x over previous
import functools

import jax
import jax.numpy as jnp
from jax.experimental import pallas as pl
from jax.experimental.pallas import tpu as pltpu


def _eca_body(w_ref, x_ref, o_ref, *, ntaps):
    x = x_ref[...]
    hw = x.shape[-1]
    mean = jnp.sum(x, axis=-1, keepdims=True, dtype=jnp.float32) * (1.0 / hw)

    pad = ntaps // 2
    acc = mean * w_ref[pad]
    for t in range(ntaps):
        d = t - pad
        if d == 0:
            continue
        if d > 0:
            shifted = jnp.concatenate(
                [mean[d:, :], jnp.zeros((d, 1), jnp.float32)], axis=0)
        else:
            shifted = jnp.concatenate(
                [jnp.zeros((-d, 1), jnp.float32), mean[:d, :]], axis=0)
        acc = acc + shifted * w_ref[t]

    scale = jax.nn.sigmoid(acc)
    o_ref[...] = x * scale.astype(x.dtype)


def kernel(x_nchw, conv_weight):
    B, C, H, W = x_nchw.shape
    HW = H * W
    x = x_nchw.reshape(B, C, HW)
    K = conv_weight.shape[0]

    out = pl.pallas_call(
        functools.partial(_eca_body, ntaps=K),
        out_shape=jax.ShapeDtypeStruct((B, C, HW), x.dtype),
        grid=(B,),
        in_specs=[
            pl.BlockSpec(memory_space=pltpu.SMEM),
            pl.BlockSpec((None, C, HW), lambda b: (b, 0, 0)),
        ],
        out_specs=pl.BlockSpec((None, C, HW), lambda b: (b, 0, 0)),
        compiler_params=pltpu.CompilerParams(
            dimension_semantics=("parallel",),
            vmem_limit_bytes=64 * 1024 * 1024,
        ),
    )(conv_weight.astype(jnp.float32), x)

    return out.reshape(B, C, H, W)
